# Initial kernel scaffold; baseline (speedup 1.0000x reference)
import jax
import jax.numpy as jnp
from jax.experimental import pallas as pl


def kernel(xi, xj, Gi2k, Gj2k, Wi, bi, Wj, bj):
    raise NotImplementedError("write your pallas kernel here")


# speedup vs baseline: 1.1311x; 1.1311x over previous
import functools

import jax
import jax.numpy as jnp
from jax.experimental import pallas as pl
from jax.experimental.pallas import tpu as pltpu

BK = 256


def _body(xi_ref, xj_ref, wi_ref, bi_ref, wj_ref, bj_ref, gi_ref, gj_ref,
          out_ref, zi_s, zj_s):
    @pl.when(pl.program_id(0) == 0)
    def _init():
        zi_s[...] = (
            jnp.dot(xi_ref[...], wi_ref[...], preferred_element_type=jnp.float32)
            + bi_ref[...]
        )
        zj_s[...] = (
            jnp.dot(xj_ref[...], wj_ref[...], preferred_element_type=jnp.float32)
            + bj_ref[...]
        )

    acc = jnp.dot(gi_ref[...], zi_s[...], preferred_element_type=jnp.float32)
    acc += jnp.dot(gj_ref[...], zj_s[...], preferred_element_type=jnp.float32)
    out_ref[...] = jnp.maximum(acc, 0.0)


@jax.jit
def kernel(xi, xj, Gi2k, Gj2k, Wi, bi, Wj, bj):
    n_k = Gi2k.shape[0]
    n_i, ci = xi.shape
    n_j, cj = xj.shape
    ck = Wi.shape[1]
    grid = (n_k // BK,)

    const = lambda shape: pl.BlockSpec(shape, lambda i: (0, 0))
    out = pl.pallas_call(
        _body,
        grid=grid,
        in_specs=[
            const((n_i, ci)),
            const((n_j, cj)),
            const((ci, ck)),
            const((1, ck)),
            const((cj, ck)),
            const((1, ck)),
            pl.BlockSpec((BK, n_i), lambda i: (i, 0)),
            pl.BlockSpec((BK, n_j), lambda i: (i, 0)),
        ],
        out_specs=pl.BlockSpec((BK, ck), lambda i: (i, 0)),
        out_shape=jax.ShapeDtypeStruct((n_k, ck), jnp.float32),
        scratch_shapes=[
            pltpu.VMEM((n_i, ck), jnp.float32),
            pltpu.VMEM((n_j, ck), jnp.float32),
        ],
        compiler_params=pltpu.CompilerParams(
            dimension_semantics=("arbitrary",),
        ),
    )(xi, xj, Wi, bi.reshape(1, ck), Wj, bj.reshape(1, ck), Gi2k, Gj2k)
    return out
